# Initial kernel scaffold; baseline (speedup 1.0000x reference)
import jax
import jax.numpy as jnp
from jax.experimental import pallas as pl


def kernel(embeddings, labels):
    raise NotImplementedError("write your pallas kernel here")


# speedup vs baseline: 10.6373x; 10.6373x over previous
import functools

import jax
import jax.numpy as jnp
from jax import lax
from jax.experimental import pallas as pl
from jax.experimental.pallas import tpu as pltpu, tpu_sc as plsc

TEMPERATURE = 0.1
EPS = 1e-07
N = 4096
NC, NS, L = 2, 16, 16
NW = NC * NS
ROWS_PER_W = N // NW
RBLK = 16
NBLK = ROWS_PER_W // RBLK
NCH = N // L
_LN2 = 0.6931471805599453


def _sims_body(e_rows_ref, e_full_ref, out_ref, aux_ref):
    ef = e_full_ref[...]
    nf = ef / jnp.maximum(jnp.sqrt(jnp.sum(ef * ef, axis=1, keepdims=True)), 1e-12)
    er = e_rows_ref[...]
    nr = er / jnp.maximum(jnp.sqrt(jnp.sum(er * er, axis=1, keepdims=True)), 1e-12)
    s = jnp.dot(nr, nf.T, preferred_element_type=jnp.float32) * (1.0 / TEMPERATURE)
    out_ref[...] = s
    bsum = jnp.sum(s)
    bdiag = jnp.sum(nr * nr) * (1.0 / TEMPERATURE)
    aux_ref[...] = jnp.full((1, 1, 128), (bsum - bdiag) * (1.0 / 128.0), jnp.float32)


def _sims(embeddings):
    n, d = embeddings.shape
    br = 512
    g = n // br
    return pl.pallas_call(
        _sims_body,
        grid=(g,),
        in_specs=[
            pl.BlockSpec((br, d), lambda i: (i, 0)),
            pl.BlockSpec((n, d), lambda i: (0, 0)),
        ],
        out_specs=[
            pl.BlockSpec((br, n), lambda i: (i, 0)),
            pl.BlockSpec((1, 1, 128), lambda i: (i, 0, 0)),
        ],
        out_shape=[
            jax.ShapeDtypeStruct((n, n), jnp.float32),
            jax.ShapeDtypeStruct((g, 1, 128), jnp.float32),
        ],
    )(embeddings, embeddings)


def _log_f32(x):
    bits = lax.bitcast_convert_type(x, jnp.int32)
    ex = (lax.shift_right_logical(bits, 23) & 255) - 127
    man = lax.bitcast_convert_type((bits & 0x007FFFFF) | 0x3F800000, jnp.float32)
    r = (man - 1.0) / (man + 1.0)
    r2 = r * r
    p = 2.0 / 9.0
    p = p * r2 + 2.0 / 7.0
    p = p * r2 + 2.0 / 5.0
    p = p * r2 + 2.0 / 3.0
    p = p * r2 + 2.0
    return ex.astype(jnp.float32) * _LN2 + r * p


def _sc_body(s_hbm, ys_hbm, p_hbm, rl_hbm, rr_hbm, part_hbm,
             ys_v, p_v, rl_v, rr_v, rows_v, e_v, q_v, r_v, idx_v, acc_v, sem):
    wid = lax.axis_index("s") * NC + lax.axis_index("c")
    lane = lax.iota(jnp.int32, L)

    for d in range(2):
        pltpu.sync_copy(ys_hbm.at[d], ys_v)
        pltpu.sync_copy(p_hbm.at[d], p_v)
        pltpu.sync_copy(rl_hbm.at[d], rl_v)
        pltpu.sync_copy(rr_hbm.at[d], rr_v)

        def blk_body(b, acc, d=d):
            base = wid * ROWS_PER_W + b * RBLK
            idx_v[...] = p_v[pl.ds(base, RBLK)]
            pltpu.async_copy(s_hbm.at[idx_v], rows_v, sem).wait()

            def row_body(r, acc):
                isr = base + r
                yi = plsc.load_gather(ys_v, [jnp.full((L,), isr, jnp.int32)])
                rvec = jnp.full((L,), r, jnp.int32)

                def p1(c, carry):
                    off = c * L
                    idxc = p_v[pl.ds(off, L)]
                    e = jnp.exp(plsc.load_gather(rows_v, [rvec, idxc]))
                    e_v[pl.ds(off, L)] = e
                    q_v[pl.ds(off, L)] = plsc.cumsum(e) + carry
                    return carry + jnp.sum(e)

                lax.fori_loop(0, NCH, p1, jnp.zeros((L,), jnp.float32))

                def p2(c2, carry):
                    off = (NCH - 1 - c2) * L
                    e = e_v[pl.ds(off, L)]
                    tot = jnp.sum(e)
                    r_v[pl.ds(off, L)] = carry + tot - plsc.cumsum(e) + e
                    return carry + tot

                lax.fori_loop(0, NCH, p2, jnp.zeros((L,), jnp.float32))

                def mn(c, acc):
                    off = c * L
                    yj = ys_v[pl.ds(off, L)]
                    rlc = rl_v[pl.ds(off, L)]
                    rrc = rr_v[pl.ds(off, L)]
                    right = yj > yi
                    tie = yj == yi
                    m = 2.0 * yi - yj
                    cnt = jnp.zeros((L,), jnp.int32)
                    for bit in (4096, 2048, 1024, 512, 256, 128, 64, 32,
                                16, 8, 4, 2, 1):
                        cand = cnt + bit
                        t = plsc.load_gather(ys_v, [jnp.minimum(cand, N) - 1])
                        ok = ((t < m) | (right & (t == m))) & (cand <= N)
                        cnt = jnp.where(ok, cand, cnt)
                    lo = jnp.where(tie, N, jnp.where(right, cnt, rrc))
                    hi = jnp.where(tie, N, jnp.where(right, rlc, cnt))
                    qv = plsc.load_gather(q_v, [jnp.maximum(lo - 1, 0)])
                    qv = jnp.where(lo > 0, qv, 0.0)
                    rv = plsc.load_gather(r_v, [jnp.minimum(hi, N - 1)])
                    rv = jnp.where(hi < N, rv, 0.0)
                    ll = _log_f32(jnp.maximum(qv + rv, EPS))
                    return acc + jnp.where(lane + off == isr, 0.0, ll)

                return lax.fori_loop(0, NCH, mn, acc)

            return lax.fori_loop(0, RBLK, row_body, acc)

        acc = lax.fori_loop(0, NBLK, blk_body, jnp.zeros((L,), jnp.float32))
        acc_v[...] = acc
        pltpu.sync_copy(acc_v, part_hbm.at[d, wid])


@functools.partial(
    pl.kernel,
    mesh=plsc.VectorSubcoreMesh(core_axis_name="c", subcore_axis_name="s"),
    out_type=jax.ShapeDtypeStruct((2, NW, L), jnp.float32),
    compiler_params=pltpu.CompilerParams(needs_layout_passes=False),
    scratch_types=[
        pltpu.VMEM((N,), jnp.float32),
        pltpu.VMEM((N,), jnp.int32),
        pltpu.VMEM((N,), jnp.int32),
        pltpu.VMEM((N,), jnp.int32),
        pltpu.VMEM((RBLK, N), jnp.float32),
        pltpu.VMEM((N,), jnp.float32),
        pltpu.VMEM((N,), jnp.float32),
        pltpu.VMEM((N,), jnp.float32),
        pltpu.VMEM((RBLK,), jnp.int32),
        pltpu.VMEM((L,), jnp.float32),
        pltpu.SemaphoreType.DMA,
    ],
)
def _sc_loss(s_hbm, ys_hbm, p_hbm, rl_hbm, rr_hbm, part_hbm, *scratch):
    _sc_body(s_hbm, ys_hbm, p_hbm, rl_hbm, rr_hbm, part_hbm, *scratch)


def kernel(embeddings, labels):
    n, _ = embeddings.shape
    assert n == N and labels.shape == (N, 2)
    sims, aux = _sims(embeddings)
    offdiag_sims = jnp.sum(aux)

    ys_l, p_l, rl_l, rr_l = [], [], [], []
    for d in range(2):
        y = labels[:, d]
        order = jnp.argsort(y)
        ys = y[order]
        ys_l.append(ys)
        p_l.append(order.astype(jnp.int32))
        rl_l.append(jnp.searchsorted(ys, ys, side="left").astype(jnp.int32))
        rr_l.append(jnp.searchsorted(ys, ys, side="right").astype(jnp.int32))

    part = _sc_loss(
        sims,
        jnp.stack(ys_l),
        jnp.stack(p_l),
        jnp.stack(rl_l),
        jnp.stack(rr_l),
    )
    log_sums = jnp.sum(part, axis=(1, 2))
    return (log_sums - offdiag_sims) / (N * (N - 1))


# speedup vs baseline: 19.3860x; 1.8224x over previous
import functools

import jax
import jax.numpy as jnp
from jax import lax
from jax.experimental import pallas as pl
from jax.experimental.pallas import tpu as pltpu, tpu_sc as plsc

TEMPERATURE = 0.1
EPS = 1e-07
N = 4096
NC, NS, L = 2, 16, 16
NW = NC * NS
ROWS_PER_W = N // NW
RBLK = 16
NBLK = ROWS_PER_W // RBLK
NCH = N // L
_LN2 = 0.6931471805599453


def _sims_body(e_rows_ref, e_full_ref, out_ref, aux_ref):
    ef = e_full_ref[...]
    nf = ef / jnp.maximum(jnp.sqrt(jnp.sum(ef * ef, axis=1, keepdims=True)), 1e-12)
    er = e_rows_ref[...]
    nr = er / jnp.maximum(jnp.sqrt(jnp.sum(er * er, axis=1, keepdims=True)), 1e-12)
    s = jnp.dot(nr, nf.T, preferred_element_type=jnp.float32) * (1.0 / TEMPERATURE)
    out_ref[...] = s
    bsum = jnp.sum(s)
    bdiag = jnp.sum(nr * nr) * (1.0 / TEMPERATURE)
    aux_ref[...] = jnp.full((1, 1, 128), (bsum - bdiag) * (1.0 / 128.0), jnp.float32)


def _sims(embeddings):
    n, d = embeddings.shape
    br = 512
    g = n // br
    return pl.pallas_call(
        _sims_body,
        grid=(g,),
        in_specs=[
            pl.BlockSpec((br, d), lambda i: (i, 0)),
            pl.BlockSpec((n, d), lambda i: (0, 0)),
        ],
        out_specs=[
            pl.BlockSpec((br, n), lambda i: (i, 0)),
            pl.BlockSpec((1, 1, 128), lambda i: (i, 0, 0)),
        ],
        out_shape=[
            jax.ShapeDtypeStruct((n, n), jnp.float32),
            jax.ShapeDtypeStruct((g, 1, 128), jnp.float32),
        ],
    )(embeddings, embeddings)


def _log_f32(x):
    bits = lax.bitcast_convert_type(x, jnp.int32)
    ex = (lax.shift_right_logical(bits, 23) & 255) - 127
    man = lax.bitcast_convert_type((bits & 0x007FFFFF) | 0x3F800000, jnp.float32)
    r = (man - 1.0) / (man + 1.0)
    r2 = r * r
    p = 2.0 / 9.0
    p = p * r2 + 2.0 / 7.0
    p = p * r2 + 2.0 / 5.0
    p = p * r2 + 2.0 / 3.0
    p = p * r2 + 2.0
    return ex.astype(jnp.float32) * _LN2 + r * p


def _sortable_key(bits):
    return jnp.where(bits >= 0, bits, jnp.int32(-2147483648) - bits)


def _sc_body(s_hbm, ys_hbm, p_hbm, rl_hbm, rr_hbm, part_hbm,
             ys_v, ysk_v, p_v, rl_v, rr_v, rows_v, e_v, q_v, r_v,
             idx_v, acc_v, sem):
    wid = lax.axis_index("s") * NC + lax.axis_index("c")
    lane = lax.iota(jnp.int32, L)

    for d in range(2):
        pltpu.sync_copy(ys_hbm.at[d], ys_v)
        pltpu.sync_copy(p_hbm.at[d], p_v)
        pltpu.sync_copy(rl_hbm.at[d], rl_v)
        pltpu.sync_copy(rr_hbm.at[d], rr_v)

        @plsc.parallel_loop(0, NCH, unroll=4)
        def _build(c):
            off = c * L
            b = lax.bitcast_convert_type(ys_v[pl.ds(off, L)], jnp.int32)
            ysk_v[pl.ds(off, L)] = _sortable_key(b)

        kmid = plsc.load_gather(ysk_v, [jnp.full((L,), 2047, jnp.int32)])
        ktop = plsc.load_gather(ysk_v, [jnp.full((L,), N - 1, jnp.int32)])

        def blk_body(b, acc, d=d):
            base = wid * ROWS_PER_W + b * RBLK
            idx_v[...] = p_v[pl.ds(base, RBLK)]
            pltpu.async_copy(s_hbm.at[idx_v], rows_v, sem).wait()

            def row_body(r, acc):
                isr = base + r
                yi = plsc.load_gather(ys_v, [jnp.full((L,), isr, jnp.int32)])
                rvec = jnp.full((L,), r, jnp.int32)

                @plsc.parallel_loop(0, NCH, unroll=4,
                                    carry=jnp.zeros((L,), jnp.float32))
                def p1(c, carry):
                    off = c * L
                    idxc = p_v[pl.ds(off, L)]
                    e = jnp.exp(plsc.load_gather(rows_v, [rvec, idxc]))
                    e_v[pl.ds(off, L)] = e
                    q_v[pl.ds(off, L)] = plsc.cumsum(e) + carry
                    return carry + jnp.sum(e)

                @plsc.parallel_loop(0, NCH, unroll=4,
                                    carry=jnp.zeros((L,), jnp.float32))
                def p2(c2, carry):
                    off = (NCH - 1 - c2) * L
                    e = e_v[pl.ds(off, L)]
                    tot = jnp.sum(e)
                    r_v[pl.ds(off, L)] = carry + tot - plsc.cumsum(e) + e
                    return carry + tot

                @plsc.parallel_loop(0, NCH, unroll=4, carry=acc)
                def mn(c, acc):
                    off = c * L
                    yj = ys_v[pl.ds(off, L)]
                    rlc = rl_v[pl.ds(off, L)]
                    rrc = rr_v[pl.ds(off, L)]
                    right = yj > yi
                    tie = yj == yi
                    m = 2.0 * yi - yj
                    mk = _sortable_key(lax.bitcast_convert_type(m, jnp.int32))
                    mk = mk + right.astype(jnp.int32)
                    cnt = jnp.where(kmid < mk, 2048, 0)
                    for bit in (1024, 512, 256, 128, 64, 32, 16, 8, 4, 2, 1):
                        cand = cnt + bit
                        t = plsc.load_gather(ysk_v, [cand - 1])
                        cnt = jnp.where(t < mk, cand, cnt)
                    cnt = jnp.where(ktop < mk, N, cnt)
                    lo = jnp.where(tie, N, jnp.where(right, cnt, rrc))
                    hi = jnp.where(tie, N, jnp.where(right, rlc, cnt))
                    qv = plsc.load_gather(q_v, [jnp.maximum(lo - 1, 0)])
                    qv = jnp.where(lo > 0, qv, 0.0)
                    rv = plsc.load_gather(r_v, [jnp.minimum(hi, N - 1)])
                    rv = jnp.where(hi < N, rv, 0.0)
                    return acc + _log_f32(jnp.maximum(qv + rv, EPS))

                stot = plsc.load_gather(q_v, [jnp.full((L,), N - 1, jnp.int32)])
                dterm = _log_f32(jnp.maximum(stot, EPS))
                return mn - jnp.where(lane == 0, dterm, 0.0)

            return lax.fori_loop(0, RBLK, row_body, acc)

        acc = lax.fori_loop(0, NBLK, blk_body, jnp.zeros((L,), jnp.float32))
        acc_v[...] = acc
        pltpu.sync_copy(acc_v, part_hbm.at[d, wid])


@functools.partial(
    pl.kernel,
    mesh=plsc.VectorSubcoreMesh(core_axis_name="c", subcore_axis_name="s"),
    out_type=jax.ShapeDtypeStruct((2, NW, L), jnp.float32),
    compiler_params=pltpu.CompilerParams(needs_layout_passes=False),
    scratch_types=[
        pltpu.VMEM((N,), jnp.float32),
        pltpu.VMEM((N,), jnp.int32),
        pltpu.VMEM((N,), jnp.int32),
        pltpu.VMEM((N,), jnp.int32),
        pltpu.VMEM((N,), jnp.int32),
        pltpu.VMEM((RBLK, N), jnp.float32),
        pltpu.VMEM((N,), jnp.float32),
        pltpu.VMEM((N,), jnp.float32),
        pltpu.VMEM((N,), jnp.float32),
        pltpu.VMEM((RBLK,), jnp.int32),
        pltpu.VMEM((L,), jnp.float32),
        pltpu.SemaphoreType.DMA,
    ],
)
def _sc_loss(s_hbm, ys_hbm, p_hbm, rl_hbm, rr_hbm, part_hbm, *scratch):
    _sc_body(s_hbm, ys_hbm, p_hbm, rl_hbm, rr_hbm, part_hbm, *scratch)


def kernel(embeddings, labels):
    n, _ = embeddings.shape
    assert n == N and labels.shape == (N, 2)
    sims, aux = _sims(embeddings)
    offdiag_sims = jnp.sum(aux)

    ys_l, p_l, rl_l, rr_l = [], [], [], []
    for d in range(2):
        y = labels[:, d]
        order = jnp.argsort(y)
        ys = y[order]
        ys_l.append(ys)
        p_l.append(order.astype(jnp.int32))
        rl_l.append(jnp.searchsorted(ys, ys, side="left").astype(jnp.int32))
        rr_l.append(jnp.searchsorted(ys, ys, side="right").astype(jnp.int32))

    part = _sc_loss(
        sims,
        jnp.stack(ys_l),
        jnp.stack(p_l),
        jnp.stack(rl_l),
        jnp.stack(rr_l),
    )
    log_sums = jnp.sum(part, axis=(1, 2))
    return (log_sums - offdiag_sims) / (N * (N - 1))


# speedup vs baseline: 22.4011x; 1.1555x over previous
import functools

import jax
import jax.numpy as jnp
from jax import lax
from jax.experimental import pallas as pl
from jax.experimental.pallas import tpu as pltpu, tpu_sc as plsc

TEMPERATURE = 0.1
EPS = 1e-07
N = 4096
NC, NS, L = 2, 16, 16
NW = NC * NS
ROWS_PER_W = N // NW
RBLK = 16
NBLK = ROWS_PER_W // RBLK
NCH = N // L
_LN2 = 0.6931471805599453


def _sims_body(e_rows_ref, e_full_ref, out_ref, aux_ref):
    ef = e_full_ref[...]
    nf = ef / jnp.maximum(jnp.sqrt(jnp.sum(ef * ef, axis=1, keepdims=True)), 1e-12)
    er = e_rows_ref[...]
    nr = er / jnp.maximum(jnp.sqrt(jnp.sum(er * er, axis=1, keepdims=True)), 1e-12)
    s = jnp.dot(nr, nf.T, preferred_element_type=jnp.float32) * (1.0 / TEMPERATURE)
    out_ref[...] = s
    bsum = jnp.sum(s)
    bdiag = jnp.sum(nr * nr) * (1.0 / TEMPERATURE)
    aux_ref[...] = jnp.full((1, 1, 128), (bsum - bdiag) * (1.0 / 128.0), jnp.float32)


def _sims(embeddings):
    n, d = embeddings.shape
    br = 512
    g = n // br
    return pl.pallas_call(
        _sims_body,
        grid=(g,),
        in_specs=[
            pl.BlockSpec((br, d), lambda i: (i, 0)),
            pl.BlockSpec((n, d), lambda i: (0, 0)),
        ],
        out_specs=[
            pl.BlockSpec((br, n), lambda i: (i, 0)),
            pl.BlockSpec((1, 1, 128), lambda i: (i, 0, 0)),
        ],
        out_shape=[
            jax.ShapeDtypeStruct((n, n), jnp.float32),
            jax.ShapeDtypeStruct((g, 1, 128), jnp.float32),
        ],
    )(embeddings, embeddings)


def _log_f32(x):
    bits = lax.bitcast_convert_type(x, jnp.int32)
    ex = (lax.shift_right_logical(bits, 23) & 255) - 127
    man = lax.bitcast_convert_type((bits & 0x007FFFFF) | 0x3F800000, jnp.float32)
    r = (man - 1.0) / (man + 1.0)
    r2 = r * r
    p = 2.0 / 9.0
    p = p * r2 + 2.0 / 7.0
    p = p * r2 + 2.0 / 5.0
    p = p * r2 + 2.0 / 3.0
    p = p * r2 + 2.0
    return ex.astype(jnp.float32) * _LN2 + r * p


def _sortable_key(bits):
    return jnp.where(bits >= 0, bits, jnp.int32(-2147483648) - bits)


def _sc_body(s_hbm, ys_hbm, p_hbm, rl_hbm, rr_hbm, part_hbm,
             ys_v, ysk_v, p_v, rl_v, rr_v, rows_v, e_v, q_v, r_v,
             idx_v, acc_v, sem):
    wid = lax.axis_index("s") * NC + lax.axis_index("c")
    lane = lax.iota(jnp.int32, L)

    for d in range(2):
        pltpu.sync_copy(ys_hbm.at[d], ys_v)
        pltpu.sync_copy(p_hbm.at[d], p_v)
        pltpu.sync_copy(rl_hbm.at[d], rl_v)
        pltpu.sync_copy(rr_hbm.at[d], rr_v)

        @plsc.parallel_loop(0, NCH, unroll=4)
        def _build(c):
            off = c * L
            b = lax.bitcast_convert_type(ys_v[pl.ds(off, L)], jnp.int32)
            ysk_v[pl.ds(off, L)] = _sortable_key(b)

        kmid = plsc.load_gather(ysk_v, [jnp.full((L,), 2047, jnp.int32)])
        ktop = plsc.load_gather(ysk_v, [jnp.full((L,), N - 1, jnp.int32)])

        def blk_body(b, acc, d=d):
            base = wid * ROWS_PER_W + b * RBLK
            idx_v[...] = p_v[pl.ds(base, RBLK)]
            pltpu.async_copy(s_hbm.at[idx_v], rows_v, sem).wait()

            def row_body(r, acc):
                isr = base + r
                yi = plsc.load_gather(ys_v, [jnp.full((L,), isr, jnp.int32)])
                rvec = jnp.full((L,), r, jnp.int32)

                @plsc.parallel_loop(0, NCH, unroll=4,
                                    carry=jnp.zeros((L,), jnp.float32))
                def p1(c, carry):
                    off = c * L
                    idxc = p_v[pl.ds(off, L)]
                    e = jnp.exp(plsc.load_gather(rows_v, [rvec, idxc]))
                    e_v[pl.ds(off, L)] = e
                    q_v[pl.ds(off, L)] = plsc.cumsum(e) + carry
                    return carry + jnp.sum(e)

                @plsc.parallel_loop(0, NCH, unroll=4,
                                    carry=jnp.zeros((L,), jnp.float32))
                def p2(c2, carry):
                    off = (NCH - 1 - c2) * L
                    e = e_v[pl.ds(off, L)]
                    tot = jnp.sum(e)
                    r_v[pl.ds(off, L)] = carry + tot - plsc.cumsum(e) + e
                    return carry + tot

                @plsc.parallel_loop(0, NCH, unroll=8, carry=acc)
                def mn(c, acc):
                    off = c * L
                    yj = ys_v[pl.ds(off, L)]
                    rlc = rl_v[pl.ds(off, L)]
                    rrc = rr_v[pl.ds(off, L)]
                    right = yj > yi
                    tie = yj == yi
                    m = 2.0 * yi - yj
                    mk = _sortable_key(lax.bitcast_convert_type(m, jnp.int32))
                    mk = mk + right.astype(jnp.int32)
                    cnt = jnp.where(kmid < mk, 2048, 0)
                    for bit in (1024, 512, 256, 128, 64, 32, 16, 8, 4, 2, 1):
                        cand = cnt + bit
                        t = plsc.load_gather(ysk_v, [cand - 1])
                        cnt = jnp.where(t < mk, cand, cnt)
                    cnt = jnp.where(ktop < mk, N, cnt)
                    lo = jnp.where(tie, N, jnp.where(right, cnt, rrc))
                    hi = jnp.where(tie, N, jnp.where(right, rlc, cnt))
                    qv = plsc.load_gather(q_v, [jnp.maximum(lo - 1, 0)])
                    qv = jnp.where(lo > 0, qv, 0.0)
                    rv = plsc.load_gather(r_v, [jnp.minimum(hi, N - 1)])
                    rv = jnp.where(hi < N, rv, 0.0)
                    return acc + _log_f32(jnp.maximum(qv + rv, EPS))

                stot = plsc.load_gather(q_v, [jnp.full((L,), N - 1, jnp.int32)])
                dterm = _log_f32(jnp.maximum(stot, EPS))
                return mn - jnp.where(lane == 0, dterm, 0.0)

            return lax.fori_loop(0, RBLK, row_body, acc)

        acc = lax.fori_loop(0, NBLK, blk_body, jnp.zeros((L,), jnp.float32))
        acc_v[...] = acc
        pltpu.sync_copy(acc_v, part_hbm.at[d, wid])


@functools.partial(
    pl.kernel,
    mesh=plsc.VectorSubcoreMesh(core_axis_name="c", subcore_axis_name="s"),
    out_type=jax.ShapeDtypeStruct((2, NW, L), jnp.float32),
    compiler_params=pltpu.CompilerParams(needs_layout_passes=False),
    scratch_types=[
        pltpu.VMEM((N,), jnp.float32),
        pltpu.VMEM((N,), jnp.int32),
        pltpu.VMEM((N,), jnp.int32),
        pltpu.VMEM((N,), jnp.int32),
        pltpu.VMEM((N,), jnp.int32),
        pltpu.VMEM((RBLK, N), jnp.float32),
        pltpu.VMEM((N,), jnp.float32),
        pltpu.VMEM((N,), jnp.float32),
        pltpu.VMEM((N,), jnp.float32),
        pltpu.VMEM((RBLK,), jnp.int32),
        pltpu.VMEM((L,), jnp.float32),
        pltpu.SemaphoreType.DMA,
    ],
)
def _sc_loss(s_hbm, ys_hbm, p_hbm, rl_hbm, rr_hbm, part_hbm, *scratch):
    _sc_body(s_hbm, ys_hbm, p_hbm, rl_hbm, rr_hbm, part_hbm, *scratch)


def kernel(embeddings, labels):
    n, _ = embeddings.shape
    assert n == N and labels.shape == (N, 2)
    sims, aux = _sims(embeddings)
    offdiag_sims = jnp.sum(aux)

    iota = lax.iota(jnp.int32, N)
    ys_l, p_l, rl_l, rr_l = [], [], [], []
    for d in range(2):
        y = labels[:, d]
        ys, order = lax.sort_key_val(y, iota)
        ys_l.append(ys)
        p_l.append(order)
        neq_prev = jnp.concatenate([jnp.ones((1,), jnp.bool_), ys[1:] != ys[:-1]])
        rl_l.append(lax.cummax(jnp.where(neq_prev, iota, 0)))
        neq_next = jnp.concatenate([ys[1:] != ys[:-1], jnp.ones((1,), jnp.bool_)])
        rr_l.append(N - jnp.flip(lax.cummax(jnp.where(jnp.flip(neq_next), iota, 0))))

    part = _sc_loss(
        sims,
        jnp.stack(ys_l),
        jnp.stack(p_l),
        jnp.stack(rl_l),
        jnp.stack(rr_l),
    )
    log_sums = jnp.sum(part, axis=(1, 2))
    return (log_sums - offdiag_sims) / (N * (N - 1))


# speedup vs baseline: 26.3419x; 1.1759x over previous
import functools

import jax
import jax.numpy as jnp
from jax import lax
from jax.experimental import pallas as pl
from jax.experimental.pallas import tpu as pltpu, tpu_sc as plsc

TEMPERATURE = 0.1
EPS = 1e-07
N = 4096
NC, NS, L = 2, 16, 16
NW = NC * NS
ROWS_PER_W = N // NW
RBLK = 16
NBLK = ROWS_PER_W // RBLK
NCH = N // L
_LN2 = 0.6931471805599453


def _sims_body(e_rows_ref, e_full_ref, out_ref, aux_ref):
    ef = e_full_ref[...]
    nf = ef / jnp.maximum(jnp.sqrt(jnp.sum(ef * ef, axis=1, keepdims=True)), 1e-12)
    er = e_rows_ref[...]
    nr = er / jnp.maximum(jnp.sqrt(jnp.sum(er * er, axis=1, keepdims=True)), 1e-12)
    s = jnp.dot(nr, nf.T, preferred_element_type=jnp.float32) * (1.0 / TEMPERATURE)
    out_ref[...] = s
    bsum = jnp.sum(s)
    bdiag = jnp.sum(nr * nr) * (1.0 / TEMPERATURE)
    aux_ref[...] = jnp.full((1, 1, 128), (bsum - bdiag) * (1.0 / 128.0), jnp.float32)


def _sims(embeddings):
    n, d = embeddings.shape
    br = 512
    g = n // br
    return pl.pallas_call(
        _sims_body,
        grid=(g,),
        in_specs=[
            pl.BlockSpec((br, d), lambda i: (i, 0)),
            pl.BlockSpec((n, d), lambda i: (0, 0)),
        ],
        out_specs=[
            pl.BlockSpec((br, n), lambda i: (i, 0)),
            pl.BlockSpec((1, 1, 128), lambda i: (i, 0, 0)),
        ],
        out_shape=[
            jax.ShapeDtypeStruct((n, n), jnp.float32),
            jax.ShapeDtypeStruct((g, 1, 128), jnp.float32),
        ],
    )(embeddings, embeddings)


def _log_f32(x):
    bits = lax.bitcast_convert_type(x, jnp.int32)
    ex = (lax.shift_right_logical(bits, 23) & 255) - 127
    man = lax.bitcast_convert_type((bits & 0x007FFFFF) | 0x3F800000, jnp.float32)
    r = (man - 1.0) / (man + 1.0)
    r2 = r * r
    p = 2.0 / 9.0
    p = p * r2 + 2.0 / 7.0
    p = p * r2 + 2.0 / 5.0
    p = p * r2 + 2.0 / 3.0
    p = p * r2 + 2.0
    return ex.astype(jnp.float32) * _LN2 + r * p


def _sortable_key(bits):
    return jnp.where(bits >= 0, bits, jnp.int32(-2147483648) - bits)


def _sc_body(s_hbm, ys_hbm, p_hbm, rl_hbm, rr_hbm, part_hbm,
             ys_v, ysk_v, p_v, rl_v, rr_v, rows_v, e_v, q_v, r_v,
             idx_v, acc_v, sem):
    wid = lax.axis_index("s") * NC + lax.axis_index("c")
    lane = lax.iota(jnp.int32, L)

    for d in range(2):
        pltpu.sync_copy(ys_hbm.at[d], ys_v)
        pltpu.sync_copy(p_hbm.at[d], p_v)
        pltpu.sync_copy(rl_hbm.at[d], rl_v)
        pltpu.sync_copy(rr_hbm.at[d], rr_v)

        @plsc.parallel_loop(0, NCH + 1, unroll=4)
        def _build(c):
            off = c * L
            src = jnp.clip(lane + (off - 1), 0, N - 1)
            b = lax.bitcast_convert_type(plsc.load_gather(ys_v, [src]), jnp.int32)
            ysk_v[pl.ds(off, L)] = _sortable_key(b)

        r_v[pl.ds(N, L)] = jnp.zeros((L,), jnp.float32)

        kmid = plsc.load_gather(ysk_v, [jnp.full((L,), 2048, jnp.int32)])
        k1q = plsc.load_gather(ysk_v, [jnp.full((L,), 1024, jnp.int32)])
        k3q = plsc.load_gather(ysk_v, [jnp.full((L,), 3072, jnp.int32)])
        ktop = plsc.load_gather(ysk_v, [jnp.full((L,), N, jnp.int32)])

        def blk_body(b, acc, d=d):
            base = wid * ROWS_PER_W + b * RBLK
            idx_v[...] = p_v[pl.ds(base, RBLK)]
            pltpu.async_copy(s_hbm.at[idx_v], rows_v, sem).wait()

            def row_body(r, acc):
                isr = base + r
                yi = plsc.load_gather(ys_v, [jnp.full((L,), isr, jnp.int32)])
                yi2 = yi + yi
                rvec = jnp.full((L,), r, jnp.int32)

                @plsc.parallel_loop(0, NCH, unroll=4,
                                    carry=jnp.zeros((L,), jnp.float32))
                def p1(c, carry):
                    off = c * L
                    idxc = p_v[pl.ds(off, L)]
                    e = jnp.exp(plsc.load_gather(rows_v, [rvec, idxc]))
                    e_v[pl.ds(off, L)] = e
                    q_v[pl.ds(off, L)] = plsc.cumsum(e) + carry - e
                    return carry + jnp.sum(e)

                q_v[pl.ds(N, L)] = p1

                @plsc.parallel_loop(0, NCH, unroll=4,
                                    carry=jnp.zeros((L,), jnp.float32))
                def p2(c2, carry):
                    off = (NCH - 1 - c2) * L
                    e = e_v[pl.ds(off, L)]
                    tot = jnp.sum(e)
                    r_v[pl.ds(off, L)] = carry + tot - plsc.cumsum(e) + e
                    return carry + tot

                @plsc.parallel_loop(0, NCH, unroll=8, carry=acc)
                def mn(c, acc):
                    off = c * L
                    yj = ys_v[pl.ds(off, L)]
                    rlc = rl_v[pl.ds(off, L)]
                    rrc = rr_v[pl.ds(off, L)]
                    ge = yj >= yi
                    m = yi2 - yj
                    mk = _sortable_key(lax.bitcast_convert_type(m, jnp.int32))
                    mk = mk + (yj > yi).astype(jnp.int32)
                    ok1 = kmid < mk
                    cnt = jnp.where(ok1, 2048, 0)
                    t2 = jnp.where(ok1, k3q, k1q)
                    cnt = jnp.where(t2 < mk, cnt + 1024, cnt)
                    for bit in (512, 256, 128, 64, 32, 16, 8, 4, 2, 1):
                        cand = cnt + bit
                        t = plsc.load_gather(ysk_v, [cand])
                        cnt = jnp.where(t < mk, cand, cnt)
                    cnt = jnp.where(ktop < mk, N, cnt)
                    lo = jnp.where(ge, cnt, rrc)
                    hi = jnp.where(ge, rlc, cnt)
                    qv = plsc.load_gather(q_v, [lo])
                    rv = plsc.load_gather(r_v, [hi])
                    return acc + _log_f32(jnp.maximum(qv + rv, EPS))

                stot = plsc.load_gather(q_v, [jnp.full((L,), N, jnp.int32)])
                dterm = _log_f32(jnp.maximum(stot, EPS))
                return mn - jnp.where(lane == 0, dterm, 0.0)

            return lax.fori_loop(0, RBLK, row_body, acc)

        acc = lax.fori_loop(0, NBLK, blk_body, jnp.zeros((L,), jnp.float32))
        acc_v[...] = acc
        pltpu.sync_copy(acc_v, part_hbm.at[d, wid])


@functools.partial(
    pl.kernel,
    mesh=plsc.VectorSubcoreMesh(core_axis_name="c", subcore_axis_name="s"),
    out_type=jax.ShapeDtypeStruct((2, NW, L), jnp.float32),
    compiler_params=pltpu.CompilerParams(needs_layout_passes=False),
    scratch_types=[
        pltpu.VMEM((N,), jnp.float32),
        pltpu.VMEM((N + L,), jnp.int32),
        pltpu.VMEM((N,), jnp.int32),
        pltpu.VMEM((N,), jnp.int32),
        pltpu.VMEM((N,), jnp.int32),
        pltpu.VMEM((RBLK, N), jnp.float32),
        pltpu.VMEM((N,), jnp.float32),
        pltpu.VMEM((N + L,), jnp.float32),
        pltpu.VMEM((N + L,), jnp.float32),
        pltpu.VMEM((RBLK,), jnp.int32),
        pltpu.VMEM((L,), jnp.float32),
        pltpu.SemaphoreType.DMA,
    ],
)
def _sc_loss(s_hbm, ys_hbm, p_hbm, rl_hbm, rr_hbm, part_hbm, *scratch):
    _sc_body(s_hbm, ys_hbm, p_hbm, rl_hbm, rr_hbm, part_hbm, *scratch)


def kernel(embeddings, labels):
    n, _ = embeddings.shape
    assert n == N and labels.shape == (N, 2)
    sims, aux = _sims(embeddings)
    offdiag_sims = jnp.sum(aux)

    iota = lax.iota(jnp.int32, N)
    ys_l, p_l, rl_l, rr_l = [], [], [], []
    for d in range(2):
        y = labels[:, d]
        ys, order = lax.sort_key_val(y, iota)
        ys_l.append(ys)
        p_l.append(order)
        neq_prev = jnp.concatenate([jnp.ones((1,), jnp.bool_), ys[1:] != ys[:-1]])
        rl_l.append(lax.cummax(jnp.where(neq_prev, iota, 0)))
        neq_next = jnp.concatenate([ys[1:] != ys[:-1], jnp.ones((1,), jnp.bool_)])
        rr_l.append(N - jnp.flip(lax.cummax(jnp.where(jnp.flip(neq_next), iota, 0))))

    part = _sc_loss(
        sims,
        jnp.stack(ys_l),
        jnp.stack(p_l),
        jnp.stack(rl_l),
        jnp.stack(rr_l),
    )
    log_sums = jnp.sum(part, axis=(1, 2))
    return (log_sums - offdiag_sims) / (N * (N - 1))


# speedup vs baseline: 28.3209x; 1.0751x over previous
import functools

import jax
import jax.numpy as jnp
from jax import lax
from jax.experimental import pallas as pl
from jax.experimental.pallas import tpu as pltpu, tpu_sc as plsc

TEMPERATURE = 0.1
EPS = 1e-07
N = 4096
NC, NS, L = 2, 16, 16
NW = NC * NS
ROWS_PER_W = N // NW
RBLK = 16
NBLK = ROWS_PER_W // RBLK
NCH = N // L
_LN2 = 0.6931471805599453


def _sims_body(e_rows_ref, e_full_ref, out_ref, aux_ref):
    ef = e_full_ref[...]
    nf = ef / jnp.maximum(jnp.sqrt(jnp.sum(ef * ef, axis=1, keepdims=True)), 1e-12)
    er = e_rows_ref[...]
    nr = er / jnp.maximum(jnp.sqrt(jnp.sum(er * er, axis=1, keepdims=True)), 1e-12)
    s = jnp.dot(nr, nf.T, preferred_element_type=jnp.float32) * (1.0 / TEMPERATURE)
    out_ref[...] = s
    bsum = jnp.sum(s)
    bdiag = jnp.sum(nr * nr) * (1.0 / TEMPERATURE)
    aux_ref[...] = jnp.full((1, 1, 128), (bsum - bdiag) * (1.0 / 128.0), jnp.float32)


def _sims(embeddings):
    n, d = embeddings.shape
    br = 512
    g = n // br
    return pl.pallas_call(
        _sims_body,
        grid=(g,),
        in_specs=[
            pl.BlockSpec((br, d), lambda i: (i, 0)),
            pl.BlockSpec((n, d), lambda i: (0, 0)),
        ],
        out_specs=[
            pl.BlockSpec((br, n), lambda i: (i, 0)),
            pl.BlockSpec((1, 1, 128), lambda i: (i, 0, 0)),
        ],
        out_shape=[
            jax.ShapeDtypeStruct((n, n), jnp.float32),
            jax.ShapeDtypeStruct((g, 1, 128), jnp.float32),
        ],
    )(embeddings, embeddings)


def _log_f32(x):
    bits = lax.bitcast_convert_type(x, jnp.int32)
    ex = (lax.shift_right_logical(bits, 23) & 255) - 127
    man = lax.bitcast_convert_type((bits & 0x007FFFFF) | 0x3F800000, jnp.float32)
    r = (man - 1.0) / (man + 1.0)
    r2 = r * r
    p = 2.0 / 9.0
    p = p * r2 + 2.0 / 7.0
    p = p * r2 + 2.0 / 5.0
    p = p * r2 + 2.0 / 3.0
    p = p * r2 + 2.0
    return ex.astype(jnp.float32) * _LN2 + r * p


def _lane_bcast(vec, idxvec):
    return lax.gather(
        vec, idxvec[:, None],
        dimension_numbers=lax.GatherDimensionNumbers(
            offset_dims=(), collapsed_slice_dims=(0,), start_index_map=(0,)),
        slice_sizes=(1,),
        mode=lax.GatherScatterMode.PROMISE_IN_BOUNDS)


def _sortable_key(bits):
    return jnp.where(bits >= 0, bits, jnp.int32(-2147483648) - bits)


def _sc_body(s_hbm, ys_hbm, p_hbm, rl_hbm, rr_hbm, part_hbm,
             ys_v, ysk_v, p_v, rl_v, rr_v, rows_v, e_v, q_v, r_v,
             idx_v, acc_v, sem):
    wid = lax.axis_index("s") * NC + lax.axis_index("c")
    lane = lax.iota(jnp.int32, L)

    for d in range(2):
        pltpu.sync_copy(ys_hbm.at[d], ys_v)
        pltpu.sync_copy(p_hbm.at[d], p_v)
        pltpu.sync_copy(rl_hbm.at[d], rl_v)
        pltpu.sync_copy(rr_hbm.at[d], rr_v)

        @plsc.parallel_loop(0, NCH + 1, unroll=4)
        def _build(c):
            off = c * L
            src = jnp.clip(lane + (off - 1), 0, N - 1)
            b = lax.bitcast_convert_type(plsc.load_gather(ys_v, [src]), jnp.int32)
            ysk_v[pl.ds(off, L)] = _sortable_key(b)

        r_v[pl.ds(N, L)] = jnp.zeros((L,), jnp.float32)

        kmid = plsc.load_gather(ysk_v, [jnp.full((L,), 2048, jnp.int32)])
        k1q = plsc.load_gather(ysk_v, [jnp.full((L,), 1024, jnp.int32)])
        k3q = plsc.load_gather(ysk_v, [jnp.full((L,), 3072, jnp.int32)])
        ktop = plsc.load_gather(ysk_v, [jnp.full((L,), N, jnp.int32)])

        def blk_body(b, acc, d=d):
            base = wid * ROWS_PER_W + b * RBLK
            idx_v[...] = p_v[pl.ds(base, RBLK)]
            pltpu.async_copy(s_hbm.at[idx_v], rows_v, sem).wait()

            def row_body(r, acc):
                isr = base + r
                yi = plsc.load_gather(ys_v, [jnp.full((L,), isr, jnp.int32)])
                yi2 = yi + yi
                rvec = jnp.full((L,), r, jnp.int32)

                lastl = jnp.full((L,), L - 1, jnp.int32)

                @plsc.parallel_loop(0, NCH, unroll=4,
                                    carry=jnp.zeros((L,), jnp.float32))
                def p1(c, carry):
                    off = c * L
                    idxc = p_v[pl.ds(off, L)]
                    e = jnp.exp(plsc.load_gather(rows_v, [rvec, idxc]))
                    e_v[pl.ds(off, L)] = e
                    cs = plsc.cumsum(e)
                    q_v[pl.ds(off, L)] = cs + carry - e
                    return carry + _lane_bcast(cs, lastl)

                q_v[pl.ds(N, L)] = p1

                @plsc.parallel_loop(0, NCH, unroll=4,
                                    carry=jnp.zeros((L,), jnp.float32))
                def p2(c2, carry):
                    off = (NCH - 1 - c2) * L
                    e = e_v[pl.ds(off, L)]
                    cs = plsc.cumsum(e)
                    tot = _lane_bcast(cs, lastl)
                    r_v[pl.ds(off, L)] = carry + tot - cs + e
                    return carry + tot

                mn_carry = (jnp.ones((L,), jnp.float32),
                            jnp.zeros((L,), jnp.int32))

                @plsc.parallel_loop(0, NCH, unroll=8, carry=mn_carry)
                def mn(c, carry):
                    prodm, eacc = carry
                    off = c * L
                    yj = ys_v[pl.ds(off, L)]
                    rlc = rl_v[pl.ds(off, L)]
                    rrc = rr_v[pl.ds(off, L)]
                    ge = yj >= yi
                    m = yi2 - yj
                    mk = _sortable_key(lax.bitcast_convert_type(m, jnp.int32))
                    mk = mk + (yj > yi).astype(jnp.int32)
                    ok1 = kmid < mk
                    cnt = jnp.where(ok1, 2048, 0)
                    t2 = jnp.where(ok1, k3q, k1q)
                    cnt = jnp.where(t2 < mk, cnt + 1024, cnt)
                    for bit in (512, 256, 128, 64, 32, 16, 8, 4, 2, 1):
                        cand = cnt + bit
                        t = plsc.load_gather(ysk_v, [cand])
                        cnt = jnp.where(t < mk, cand, cnt)
                    cnt = jnp.where(ktop < mk, N, cnt)
                    lo = jnp.where(ge, cnt, rrc)
                    hi = jnp.where(ge, rlc, cnt)
                    qv = plsc.load_gather(q_v, [lo])
                    rv = plsc.load_gather(r_v, [hi])
                    dbits = lax.bitcast_convert_type(
                        jnp.maximum(qv + rv, EPS), jnp.int32)
                    eacc = eacc + (lax.shift_right_logical(dbits, 23) & 255)
                    man = lax.bitcast_convert_type(
                        (dbits & 0x007FFFFF) | 0x3F800000, jnp.float32)
                    t = prodm * man
                    big = t >= 2.0
                    prodm = jnp.where(big, t * 0.5, t)
                    eacc = eacc + big.astype(jnp.int32)
                    return prodm, eacc

                prodm, eacc = mn
                stot = plsc.load_gather(q_v, [jnp.full((L,), N, jnp.int32)])
                dterm = _log_f32(jnp.maximum(stot, EPS))
                row_log = (_log_f32(prodm)
                           + (eacc.astype(jnp.float32) - 127.0 * NCH) * _LN2)
                return acc + row_log - jnp.where(lane == 0, dterm, 0.0)

            return lax.fori_loop(0, RBLK, row_body, acc)

        acc = lax.fori_loop(0, NBLK, blk_body, jnp.zeros((L,), jnp.float32))
        acc_v[...] = acc
        pltpu.sync_copy(acc_v, part_hbm.at[d, wid])


@functools.partial(
    pl.kernel,
    mesh=plsc.VectorSubcoreMesh(core_axis_name="c", subcore_axis_name="s"),
    out_type=jax.ShapeDtypeStruct((2, NW, L), jnp.float32),
    compiler_params=pltpu.CompilerParams(needs_layout_passes=False),
    scratch_types=[
        pltpu.VMEM((N,), jnp.float32),
        pltpu.VMEM((N + L,), jnp.int32),
        pltpu.VMEM((N,), jnp.int32),
        pltpu.VMEM((N,), jnp.int32),
        pltpu.VMEM((N,), jnp.int32),
        pltpu.VMEM((RBLK, N), jnp.float32),
        pltpu.VMEM((N,), jnp.float32),
        pltpu.VMEM((N + L,), jnp.float32),
        pltpu.VMEM((N + L,), jnp.float32),
        pltpu.VMEM((RBLK,), jnp.int32),
        pltpu.VMEM((L,), jnp.float32),
        pltpu.SemaphoreType.DMA,
    ],
)
def _sc_loss(s_hbm, ys_hbm, p_hbm, rl_hbm, rr_hbm, part_hbm, *scratch):
    _sc_body(s_hbm, ys_hbm, p_hbm, rl_hbm, rr_hbm, part_hbm, *scratch)


def kernel(embeddings, labels):
    n, _ = embeddings.shape
    assert n == N and labels.shape == (N, 2)
    sims, aux = _sims(embeddings)
    offdiag_sims = jnp.sum(aux)

    iota = lax.iota(jnp.int32, N)
    ys_l, p_l, rl_l, rr_l = [], [], [], []
    for d in range(2):
        y = labels[:, d]
        ys, order = lax.sort_key_val(y, iota)
        ys_l.append(ys)
        p_l.append(order)
        neq_prev = jnp.concatenate([jnp.ones((1,), jnp.bool_), ys[1:] != ys[:-1]])
        rl_l.append(lax.cummax(jnp.where(neq_prev, iota, 0)))
        neq_next = jnp.concatenate([ys[1:] != ys[:-1], jnp.ones((1,), jnp.bool_)])
        rr_l.append(N - jnp.flip(lax.cummax(jnp.where(jnp.flip(neq_next), iota, 0))))

    part = _sc_loss(
        sims,
        jnp.stack(ys_l),
        jnp.stack(p_l),
        jnp.stack(rl_l),
        jnp.stack(rr_l),
    )
    log_sums = jnp.sum(part, axis=(1, 2))
    return (log_sums - offdiag_sims) / (N * (N - 1))


# speedup vs baseline: 29.1339x; 1.0287x over previous
import functools

import jax
import jax.numpy as jnp
from jax import lax
from jax.experimental import pallas as pl
from jax.experimental.pallas import tpu as pltpu, tpu_sc as plsc

TEMPERATURE = 0.1
EPS = 1e-07
N = 4096
NC, NS, L = 2, 16, 16
NW = NC * NS
ROWS_PER_W = N // NW
RBLK = 16
NBLK = ROWS_PER_W // RBLK
NCH = N // L
_LN2 = 0.6931471805599453


def _sims_body(e_rows_ref, e_full_ref, out_ref, aux_ref):
    ef = e_full_ref[...]
    nf = ef / jnp.maximum(jnp.sqrt(jnp.sum(ef * ef, axis=1, keepdims=True)), 1e-12)
    er = e_rows_ref[...]
    nr = er / jnp.maximum(jnp.sqrt(jnp.sum(er * er, axis=1, keepdims=True)), 1e-12)
    s = jnp.dot(nr, nf.T, preferred_element_type=jnp.float32) * (1.0 / TEMPERATURE)
    out_ref[...] = s
    bsum = jnp.sum(s)
    bdiag = jnp.sum(nr * nr) * (1.0 / TEMPERATURE)
    aux_ref[...] = jnp.full((1, 1, 128), (bsum - bdiag) * (1.0 / 128.0), jnp.float32)


def _sims(embeddings):
    n, d = embeddings.shape
    br = 512
    g = n // br
    return pl.pallas_call(
        _sims_body,
        grid=(g,),
        in_specs=[
            pl.BlockSpec((br, d), lambda i: (i, 0)),
            pl.BlockSpec((n, d), lambda i: (0, 0)),
        ],
        out_specs=[
            pl.BlockSpec((br, n), lambda i: (i, 0)),
            pl.BlockSpec((1, 1, 128), lambda i: (i, 0, 0)),
        ],
        out_shape=[
            jax.ShapeDtypeStruct((n, n), jnp.float32),
            jax.ShapeDtypeStruct((g, 1, 128), jnp.float32),
        ],
    )(embeddings, embeddings)


def _log_f32(x):
    bits = lax.bitcast_convert_type(x, jnp.int32)
    ex = (lax.shift_right_logical(bits, 23) & 255) - 127
    man = lax.bitcast_convert_type((bits & 0x007FFFFF) | 0x3F800000, jnp.float32)
    r = (man - 1.0) / (man + 1.0)
    r2 = r * r
    p = 2.0 / 9.0
    p = p * r2 + 2.0 / 7.0
    p = p * r2 + 2.0 / 5.0
    p = p * r2 + 2.0 / 3.0
    p = p * r2 + 2.0
    return ex.astype(jnp.float32) * _LN2 + r * p


def _lane_bcast(vec, idxvec):
    return lax.gather(
        vec, idxvec[:, None],
        dimension_numbers=lax.GatherDimensionNumbers(
            offset_dims=(), collapsed_slice_dims=(0,), start_index_map=(0,)),
        slice_sizes=(1,),
        mode=lax.GatherScatterMode.PROMISE_IN_BOUNDS)


def _sortable_key(bits):
    return jnp.where(bits >= 0, bits, jnp.int32(-2147483648) - bits)


def _sc_body(s0_hbm, s1_hbm, ys_hbm, rl_hbm, rr_hbm, part_hbm,
             ys_v, ysk_v, rl_v, rr_v, rows_v, e_v, q_v, r_v,
             acc_v, sem):
    wid = lax.axis_index("s") * NC + lax.axis_index("c")
    lane = lax.iota(jnp.int32, L)

    for d, sd_hbm in enumerate((s0_hbm, s1_hbm)):
        pltpu.sync_copy(ys_hbm.at[d], ys_v)
        pltpu.sync_copy(rl_hbm.at[d], rl_v)
        pltpu.sync_copy(rr_hbm.at[d], rr_v)

        @plsc.parallel_loop(0, NCH + 1, unroll=4)
        def _build(c):
            off = c * L
            src = jnp.clip(lane + (off - 1), 0, N - 1)
            b = lax.bitcast_convert_type(plsc.load_gather(ys_v, [src]), jnp.int32)
            ysk_v[pl.ds(off, L)] = _sortable_key(b)

        r_v[pl.ds(N, L)] = jnp.zeros((L,), jnp.float32)

        kmid = plsc.load_gather(ysk_v, [jnp.full((L,), 2048, jnp.int32)])
        k1q = plsc.load_gather(ysk_v, [jnp.full((L,), 1024, jnp.int32)])
        k3q = plsc.load_gather(ysk_v, [jnp.full((L,), 3072, jnp.int32)])
        ktop = plsc.load_gather(ysk_v, [jnp.full((L,), N, jnp.int32)])

        def blk_body(b, acc, sd_hbm=sd_hbm):
            base = wid * ROWS_PER_W + b * RBLK
            pltpu.async_copy(sd_hbm.at[pl.ds(base, RBLK)], rows_v, sem).wait()

            def row_body(r, acc):
                isr = base + r
                yi = plsc.load_gather(ys_v, [jnp.full((L,), isr, jnp.int32)])
                yi2 = yi + yi

                lastl = jnp.full((L,), L - 1, jnp.int32)

                @plsc.parallel_loop(0, NCH, unroll=4,
                                    carry=jnp.zeros((L,), jnp.float32))
                def p1(c, carry):
                    off = c * L
                    e = jnp.exp(rows_v[r, pl.ds(off, L)])
                    e_v[pl.ds(off, L)] = e
                    cs = plsc.cumsum(e)
                    q_v[pl.ds(off, L)] = cs + carry - e
                    return carry + _lane_bcast(cs, lastl)

                q_v[pl.ds(N, L)] = p1

                @plsc.parallel_loop(0, NCH, unroll=4,
                                    carry=jnp.zeros((L,), jnp.float32))
                def p2(c2, carry):
                    off = (NCH - 1 - c2) * L
                    e = e_v[pl.ds(off, L)]
                    cs = plsc.cumsum(e)
                    tot = _lane_bcast(cs, lastl)
                    r_v[pl.ds(off, L)] = carry + tot - cs + e
                    return carry + tot

                mn_carry = (jnp.ones((L,), jnp.float32),
                            jnp.zeros((L,), jnp.int32))

                @plsc.parallel_loop(0, NCH, unroll=8, carry=mn_carry)
                def mn(c, carry):
                    prodm, eacc = carry
                    off = c * L
                    yj = ys_v[pl.ds(off, L)]
                    rlc = rl_v[pl.ds(off, L)]
                    rrc = rr_v[pl.ds(off, L)]
                    ge = yj >= yi
                    m = yi2 - yj
                    mk = _sortable_key(lax.bitcast_convert_type(m, jnp.int32))
                    mk = mk + (yj > yi).astype(jnp.int32)
                    ok1 = kmid < mk
                    cnt = jnp.where(ok1, 2048, 0)
                    t2 = jnp.where(ok1, k3q, k1q)
                    cnt = jnp.where(t2 < mk, cnt + 1024, cnt)
                    for bit in (512, 256, 128, 64, 32, 16, 8, 4, 2, 1):
                        cand = cnt + bit
                        t = plsc.load_gather(ysk_v, [cand])
                        cnt = jnp.where(t < mk, cand, cnt)
                    cnt = jnp.where(ktop < mk, N, cnt)
                    lo = jnp.where(ge, cnt, rrc)
                    hi = jnp.where(ge, rlc, cnt)
                    qv = plsc.load_gather(q_v, [lo])
                    rv = plsc.load_gather(r_v, [hi])
                    dbits = lax.bitcast_convert_type(
                        jnp.maximum(qv + rv, EPS), jnp.int32)
                    eacc = eacc + (lax.shift_right_logical(dbits, 23) & 255)
                    man = lax.bitcast_convert_type(
                        (dbits & 0x007FFFFF) | 0x3F800000, jnp.float32)
                    t = prodm * man
                    big = t >= 2.0
                    prodm = jnp.where(big, t * 0.5, t)
                    eacc = eacc + big.astype(jnp.int32)
                    return prodm, eacc

                prodm, eacc = mn
                stot = plsc.load_gather(q_v, [jnp.full((L,), N, jnp.int32)])
                dterm = _log_f32(jnp.maximum(stot, EPS))
                row_log = (_log_f32(prodm)
                           + (eacc.astype(jnp.float32) - 127.0 * NCH) * _LN2)
                return acc + row_log - jnp.where(lane == 0, dterm, 0.0)

            return lax.fori_loop(0, RBLK, row_body, acc)

        acc = lax.fori_loop(0, NBLK, blk_body, jnp.zeros((L,), jnp.float32))
        acc_v[...] = acc
        pltpu.sync_copy(acc_v, part_hbm.at[d, wid])


@functools.partial(
    pl.kernel,
    mesh=plsc.VectorSubcoreMesh(core_axis_name="c", subcore_axis_name="s"),
    out_type=jax.ShapeDtypeStruct((2, NW, L), jnp.float32),
    compiler_params=pltpu.CompilerParams(needs_layout_passes=False),
    scratch_types=[
        pltpu.VMEM((N,), jnp.float32),
        pltpu.VMEM((N + L,), jnp.int32),
        pltpu.VMEM((N,), jnp.int32),
        pltpu.VMEM((N,), jnp.int32),
        pltpu.VMEM((RBLK, N), jnp.float32),
        pltpu.VMEM((N,), jnp.float32),
        pltpu.VMEM((N + L,), jnp.float32),
        pltpu.VMEM((N + L,), jnp.float32),
        pltpu.VMEM((L,), jnp.float32),
        pltpu.SemaphoreType.DMA,
    ],
)
def _sc_loss(s0_hbm, s1_hbm, ys_hbm, rl_hbm, rr_hbm, part_hbm, *scratch):
    _sc_body(s0_hbm, s1_hbm, ys_hbm, rl_hbm, rr_hbm, part_hbm, *scratch)


def kernel(embeddings, labels):
    n, _ = embeddings.shape
    assert n == N and labels.shape == (N, 2)

    iota = lax.iota(jnp.int32, N)
    sims_l, ys_l, rl_l, rr_l = [], [], [], []
    aux = None
    for d in range(2):
        y = labels[:, d]
        ys, order = lax.sort_key_val(y, iota)
        ys_l.append(ys)
        sims_d, aux_d = _sims(embeddings[order])
        sims_l.append(sims_d)
        if aux is None:
            aux = aux_d
        neq_prev = jnp.concatenate([jnp.ones((1,), jnp.bool_), ys[1:] != ys[:-1]])
        rl_l.append(lax.cummax(jnp.where(neq_prev, iota, 0)))
        neq_next = jnp.concatenate([ys[1:] != ys[:-1], jnp.ones((1,), jnp.bool_)])
        rr_l.append(N - jnp.flip(lax.cummax(jnp.where(jnp.flip(neq_next), iota, 0))))

    offdiag_sims = jnp.sum(aux)
    part = _sc_loss(
        sims_l[0],
        sims_l[1],
        jnp.stack(ys_l),
        jnp.stack(rl_l),
        jnp.stack(rr_l),
    )
    log_sums = jnp.sum(part, axis=(1, 2))
    return (log_sums - offdiag_sims) / (N * (N - 1))
